# baseline (speedup 1.0000x reference)
import functools

import jax
import jax.numpy as jnp
from jax import lax
from jax.experimental import pallas as pl
from jax.experimental.pallas import tpu as pltpu
from jax.experimental.pallas import tpu_sc as plsc


def _scores_kernel(shift_ref, qkvp_ref, idx_ref, pv_ref,
                   s_ref, idxo_ref, pvo_ref, *, c, H, dh, scale, n, topk):
    blk = qkvp_ref[0]
    for h in range(H):
        q = blk[:, h * dh:(h + 1) * dh]
        k = blk[:, c + h * dh:c + (h + 1) * dh]
        for par in range(2):
            kp = k[par * 128:(par + 1) * 128, :]
            s = lax.dot_general(q, kp, (((1,), (1,)), ((), ())),
                                preferred_element_type=jnp.float32)
            s_ref[h, par] = s * scale
    for h in range(H):
        for m in range(128 // topk):
            idxo_ref[h, :, m * topk:(m + 1) * topk] = (
                idx_ref[0, 0, h, m::(128 // topk), :])
            pvo_ref[h, :, m * topk:(m + 1) * topk] = (
                pv_ref[0, 0, h, m::(128 // topk), :])


def _make_scores(bc, boff, n, c, H, dh, topk, scale):
    nr = (n * topk) // 128
    return pl.pallas_call(
        functools.partial(_scores_kernel, c=c, H=H, dh=dh, scale=scale,
                          n=n, topk=topk),
        grid_spec=pltpu.PrefetchScalarGridSpec(
            num_scalar_prefetch=1,
            grid=(bc,),
            in_specs=[
                pl.BlockSpec((1, n, 2 * c), lambda b, sref: (b + boff, 0, 0)),
                pl.BlockSpec((1, 1, H, n, topk),
                             lambda b, sref: (sref[0], b + boff, 0, 0, 0)),
                pl.BlockSpec((1, 1, H, n, topk),
                             lambda b, sref: (sref[0], b + boff, 0, 0, 0)),
            ],
            out_specs=[
                pl.BlockSpec((H, 2, n, 128), lambda b, sref: (b, 0, 0, 0)),
                pl.BlockSpec((H, nr, 128), lambda b, sref: (b, 0, 0)),
                pl.BlockSpec((H, nr, 128), lambda b, sref: (b, 0, 0)),
            ],
        ),
        out_shape=[
            jax.ShapeDtypeStruct((bc * H, 2, n, 128), jnp.float32),
            jax.ShapeDtypeStruct((bc * H, nr, 128), jnp.int32),
            jax.ShapeDtypeStruct((bc * H, nr, 128), jnp.float32),
        ],
    )


def _sc_attn_body(s_hbm, idx_hbm, pv_hbm, rpi_hbm, rpb_hbm, w_hbm,
                  s_v0, s_v1, s_v2, s_v3,
                  ix0, ix1, ix2, ix3, ix4, ix5, ix6, ix7,
                  pv_v0, pv_v1, pv_v2, pv_v3,
                  w_v0, w_v1, w_v2, w_v3, rpi_v, rpb_v, r_v,
                  sem_s0, sem_s1, sem_s2, sem_s3,
                  sem_i0, sem_i1, sem_i2, sem_i3,
                  sem_p0, sem_p1, sem_p2, sem_p3,
                  sem_o0, sem_o1, sem_o2, sem_o3,
                  *, b_, n, H, topk, rows):
    cid = lax.axis_index("c")
    sid = lax.axis_index("s")
    wid = sid * 2 + cid
    i0 = wid * rows

    s_vs = [s_v0, s_v1, s_v2, s_v3]
    idx_vs = [ix0, ix1, ix2, ix3, ix4, ix5, ix6, ix7]
    pv_vs = [pv_v0, pv_v1, pv_v2, pv_v3]
    w_vs = [w_v0, w_v1, w_v2, w_v3]
    sem_s = [sem_s0, sem_s1, sem_s2, sem_s3]
    sem_i = [sem_i0, sem_i1, sem_i2, sem_i3]
    sem_p = [sem_p0, sem_p1, sem_p2, sem_p3]
    sem_o = [sem_o0, sem_o1, sem_o2, sem_o3]

    pltpu.sync_copy(rpi_hbm.at[pl.ds(i0 * 2, rows * 2), :], rpi_v)
    pltpu.sync_copy(rpb_hbm, rpb_v)

    for h in range(H):
        h16 = jnp.full((16,), h, jnp.int32)
        for rr in range(2 * rows):
            for g in range(128 // 16):
                ri = rpi_v[rr, g * 16:(g + 1) * 16]
                f = ri * H + h16
                rb = plsc.load_gather(
                    rpb_v, [lax.shift_right_logical(f, 7),
                            lax.bitwise_and(f, jnp.full((16,), 127,
                                                        jnp.int32))])
                r_v[h * 2 * rows + rr, g * 16:(g + 1) * 16] = rb

    zeros16 = jnp.zeros((16,), jnp.float32)
    for p in range(4):
        for par in range(2):
            for r in range(rows):
                for j0 in range(0, 128, 16):
                    w_vs[p][par, r, j0:j0 + 16] = zeros16

    ngroups = topk // 16
    nbh = b_ * H
    ir0 = (i0 * topk) // 128
    irows = (rows * topk) // 128

    def slc(buf, r, g):
        flat = r * topk + g * 16
        return buf[flat // 128, flat % 128:flat % 128 + 16]

    D = 4

    def fire_inputs(bh, q8):
        q = q8 % D
        pltpu.async_copy(s_hbm.at[bh, :, pl.ds(i0, rows), :], s_vs[q],
                         sem_s[q])
        pltpu.async_copy(idx_hbm.at[bh, pl.ds(ir0, irows), :],
                         idx_vs[q8], sem_i[q])
        pltpu.async_copy(pv_hbm.at[bh, pl.ds(ir0, irows), :],
                         pv_vs[q], sem_p[q])

    def wait_inputs(bh, q8):
        q = q8 % D
        pltpu.make_async_copy(s_hbm.at[bh, :, pl.ds(i0, rows), :], s_vs[q],
                              sem_s[q]).wait()
        pltpu.make_async_copy(idx_hbm.at[bh, pl.ds(ir0, irows), :],
                              idx_vs[q8], sem_i[q]).wait()
        pltpu.make_async_copy(pv_hbm.at[bh, pl.ds(ir0, irows), :],
                              pv_vs[q], sem_p[q]).wait()

    for bh0 in range(D):
        fire_inputs(bh0, bh0)

    c127 = jnp.full((16,), 127, jnp.int32)

    def bh_iter(bh, q8):
        q = q8 % D
        h = bh % H

        @pl.when(bh >= D)
        def _drain_and_rezero():
            pltpu.make_async_copy(w_vs[q],
                                  w_hbm.at[bh - D, :, pl.ds(i0, rows), :],
                                  sem_o[q]).wait()
            oidx = idx_vs[(q8 + D) % 8]
            for r in range(rows):
                r16 = jnp.full((16,), r, jnp.int32)
                for g in range(ngroups):
                    og = slc(oidx, r, g)
                    plsc.store_scatter(
                        w_vs[q],
                        [lax.shift_right_logical(og, 7), r16,
                         lax.bitwise_and(og, c127)],
                        zeros16)

        wait_inputs(bh, q8)
        hbase = jnp.full((16,), h * 2 * rows, jnp.int32)
        for r in range(rows):
            r16 = jnp.full((16,), r, jnp.int32)
            rbase = hbase + (2 * r)
            for g in range(ngroups):
                idx_g = slc(idx_vs[q8], r, g)
                jhi = lax.shift_right_logical(idx_g, 7)
                jlo = lax.bitwise_and(idx_g, c127)
                s_g = plsc.load_gather(s_vs[q], [jhi, r16, jlo])
                rb_g = plsc.load_gather(r_v, [rbase + jhi, jlo])
                e_g = jnp.exp(s_g + rb_g) * slc(pv_vs[q], r, g)
                plsc.addupdate_scatter(w_vs[q], [jhi, r16, jlo], e_g)
        pltpu.async_copy(w_vs[q], w_hbm.at[bh, :, pl.ds(i0, rows), :],
                         sem_o[q])

        @pl.when(bh + D < nbh)
        def _prefetch():
            fire_inputs(bh + D, (q8 + D) % 8)

    def outer(t, carry):
        for qq in range(8):
            bh_iter(t * 8 + qq, qq)
        return carry

    lax.fori_loop(0, nbh // 8, outer, 0)

    for dd in range(D):
        bh = nbh - D + dd
        pltpu.make_async_copy(w_vs[bh % D],
                              w_hbm.at[bh, :, pl.ds(i0, rows), :],
                              sem_o[bh % D]).wait()


def _make_sc_attn(b_, n, H, topk, nrpbf):
    rows = n // 32
    irows = (rows * topk) // 128
    mesh = plsc.VectorSubcoreMesh(core_axis_name="c", subcore_axis_name="s",
                                  num_cores=2, num_subcores=16)
    idx_buf = pltpu.VMEM((irows, 128), jnp.int32)
    pv_buf = pltpu.VMEM((irows, 128), jnp.float32)
    return pl.kernel(
        functools.partial(_sc_attn_body, b_=b_, n=n, H=H, topk=topk,
                          rows=rows),
        out_type=jax.ShapeDtypeStruct((b_ * H, 2, n, 128), jnp.float32),
        mesh=mesh,
        compiler_params=pltpu.CompilerParams(use_tc_tiling_on_sc=False,
                                             needs_layout_passes=False),
        scratch_types=(
            [pltpu.VMEM((2, rows, 128), jnp.float32)] * 4 +
            [idx_buf] * 8 +
            [pv_buf] * 4 +
            [pltpu.VMEM((2, rows, 128), jnp.float32)] * 4 +
            [
                pltpu.VMEM((2 * rows, 128), jnp.int32),
                pltpu.VMEM((nrpbf, 128), jnp.float32),
                pltpu.VMEM((H * 2 * rows, 128), jnp.float32),
            ] +
            [pltpu.SemaphoreType.DMA] * 16
        ),
    )


def _out_kernel(w_ref, qkvp_ref, pw_ref, pb_ref, o_ref, *, c, H, dh):
    blk = qkvp_ref[0]
    parts = []
    for h in range(H):
        v = blk[:, h * dh:(h + 1) * dh]
        w0 = w_ref[h, 0]
        w1 = w_ref[h, 1]
        den = (jnp.sum(w0, axis=1, keepdims=True) +
               jnp.sum(w1, axis=1, keepdims=True) + 1e-20)
        xh = lax.dot_general(w0, v[0:128, :], (((1,), (0,)), ((), ())),
                             preferred_element_type=jnp.float32)
        xh = xh + lax.dot_general(w1, v[128:256, :],
                                  (((1,), (0,)), ((), ())),
                                  preferred_element_type=jnp.float32)
        parts.append(xh / den)
    x = jnp.concatenate(parts, axis=1) + blk[:, c:2 * c]
    o = lax.dot_general(x, pw_ref[...], (((1,), (1,)), ((), ())),
                        preferred_element_type=jnp.float32)
    o_ref[0] = o + pb_ref[...]


def _make_out(bc, boff, n, c, H, dh):
    return pl.pallas_call(
        functools.partial(_out_kernel, c=c, H=H, dh=dh),
        grid=(bc,),
        in_specs=[
            pl.BlockSpec((H, 2, n, 128), lambda b: (b, 0, 0, 0)),
            pl.BlockSpec((1, n, 2 * c), lambda b: (b + boff, 0, 1)),
            pl.BlockSpec((c, c), lambda b: (0, 0)),
            pl.BlockSpec((1, c), lambda b: (0, 0)),
        ],
        out_specs=pl.BlockSpec((1, n, c), lambda b: (b, 0, 0)),
        out_shape=jax.ShapeDtypeStruct((bc, n, c), jnp.float32),
    )


def kernel(qkvp, pfa_values, pfa_indices, rpi, rpb_table, proj_w, proj_b,
           shift):
    b_, n, c4 = qkvp.shape
    c = c4 // 4
    H = rpb_table.shape[1]
    dh = c // H
    topk = pfa_indices.shape[-1]
    nrpb = rpb_table.shape[0]
    nrpbf = -(-(nrpb * H) // 128)
    scale = dh ** (-0.5)

    shift_arr = jnp.asarray(shift, jnp.int32).reshape(1)
    rpir = rpi.astype(jnp.int32).reshape((n * n) // 128, 128)
    rpbf = jnp.pad(rpb_table.reshape(-1),
                   (0, nrpbf * 128 - nrpb * H)).reshape(nrpbf, 128)
    idx32 = pfa_indices.astype(jnp.int32)
    pb2 = proj_b.reshape(1, c)

    nchunks = 4
    bc = b_ // nchunks
    outs = []
    for ci in range(nchunks):
        boff = ci * bc
        s, idxr, pvr = _make_scores(bc, boff, n, c, H, dh, topk, scale)(
            shift_arr, qkvp, idx32, pfa_values)
        w = _make_sc_attn(bc, n, H, topk, nrpbf)(s, idxr, pvr, rpir, rpbf)
        outs.append(_make_out(bc, boff, n, c, H, dh)(w, qkvp, proj_w, pb2))
    return jnp.concatenate(outs, axis=0)


# speedup vs baseline: 1.0725x; 1.0725x over previous
import functools

import jax
import jax.numpy as jnp
from jax import lax
from jax.experimental import pallas as pl
from jax.experimental.pallas import tpu as pltpu
from jax.experimental.pallas import tpu_sc as plsc


def _scores_kernel(shift_ref, qkvp_ref, idx_ref, pv_ref,
                   s_ref, io_ref, *, c, H, dh, scale, n, topk):
    blk = qkvp_ref[0]
    for h in range(H):
        q = blk[:, h * dh:(h + 1) * dh]
        k = blk[:, c + h * dh:c + (h + 1) * dh]
        for par in range(2):
            kp = k[par * 128:(par + 1) * 128, :]
            s = lax.dot_general(q, kp, (((1,), (1,)), ((), ())),
                                preferred_element_type=jnp.float32)
            s_ref[h, par] = s * scale
    for h in range(H):
        for m in range(128 // topk):
            io_ref[h, 0, :, m * topk:(m + 1) * topk] = (
                idx_ref[0, 0, h, m::(128 // topk), :])
            io_ref[h, 1, :, m * topk:(m + 1) * topk] = (
                lax.bitcast_convert_type(
                    pv_ref[0, 0, h, m::(128 // topk), :], jnp.int32))


def _make_scores(bc, boff, n, c, H, dh, topk, scale):
    nr = (n * topk) // 128
    return pl.pallas_call(
        functools.partial(_scores_kernel, c=c, H=H, dh=dh, scale=scale,
                          n=n, topk=topk),
        grid_spec=pltpu.PrefetchScalarGridSpec(
            num_scalar_prefetch=1,
            grid=(bc,),
            in_specs=[
                pl.BlockSpec((1, n, 2 * c), lambda b, sref: (b + boff, 0, 0)),
                pl.BlockSpec((1, 1, H, n, topk),
                             lambda b, sref: (sref[0], b + boff, 0, 0, 0)),
                pl.BlockSpec((1, 1, H, n, topk),
                             lambda b, sref: (sref[0], b + boff, 0, 0, 0)),
            ],
            out_specs=[
                pl.BlockSpec((H, 2, n, 128), lambda b, sref: (b, 0, 0, 0)),
                pl.BlockSpec((H, 2, nr, 128), lambda b, sref: (b, 0, 0, 0)),
            ],
        ),
        out_shape=[
            jax.ShapeDtypeStruct((bc * H, 2, n, 128), jnp.float32),
            jax.ShapeDtypeStruct((bc * H, 2, nr, 128), jnp.int32),
        ],
    )


def _sc_attn_body(s_hbm, ipv_hbm, rpi_hbm, rpb_hbm, w_hbm,
                  s_v0, s_v1, s_v2, s_v3,
                  ix0, ix1, ix2, ix3, ix4, ix5, ix6, ix7,
                  w_v0, w_v1, w_v2, w_v3, rpi_v, rpb_v, r_v,
                  sem_s0, sem_s1, sem_s2, sem_s3,
                  sem_i0, sem_i1, sem_i2, sem_i3,
                  sem_o0, sem_o1, sem_o2, sem_o3,
                  *, b_, n, H, topk, rows):
    cid = lax.axis_index("c")
    sid = lax.axis_index("s")
    wid = sid * 2 + cid
    i0 = wid * rows

    s_vs = [s_v0, s_v1, s_v2, s_v3]
    ipv_vs = [ix0, ix1, ix2, ix3, ix4, ix5, ix6, ix7]
    w_vs = [w_v0, w_v1, w_v2, w_v3]
    sem_s = [sem_s0, sem_s1, sem_s2, sem_s3]
    sem_i = [sem_i0, sem_i1, sem_i2, sem_i3]
    sem_o = [sem_o0, sem_o1, sem_o2, sem_o3]

    pltpu.sync_copy(rpi_hbm.at[pl.ds(i0 * 2, rows * 2), :], rpi_v)
    pltpu.sync_copy(rpb_hbm, rpb_v)

    for h in range(H):
        h16 = jnp.full((16,), h, jnp.int32)
        for rr in range(2 * rows):
            for g in range(128 // 16):
                ri = rpi_v[rr, g * 16:(g + 1) * 16]
                f = ri * H + h16
                rb = plsc.load_gather(
                    rpb_v, [lax.shift_right_logical(f, 7),
                            lax.bitwise_and(f, jnp.full((16,), 127,
                                                        jnp.int32))])
                r_v[h * 2 * rows + rr, g * 16:(g + 1) * 16] = rb

    zeros16 = jnp.zeros((16,), jnp.float32)
    for p in range(4):
        for par in range(2):
            for r in range(rows):
                for j0 in range(0, 128, 16):
                    w_vs[p][par, r, j0:j0 + 16] = zeros16

    ngroups = topk // 16
    nbh = b_ * H
    ir0 = (i0 * topk) // 128
    irows = (rows * topk) // 128

    def slc(buf, pl_, r, g):
        flat = r * topk + g * 16
        return buf[pl_, flat // 128, flat % 128:flat % 128 + 16]

    D = 4

    def fire_inputs(bh, q8):
        q = q8 % D
        pltpu.async_copy(s_hbm.at[bh, :, pl.ds(i0, rows), :], s_vs[q],
                         sem_s[q])
        pltpu.async_copy(ipv_hbm.at[bh, :, pl.ds(ir0, irows), :],
                         ipv_vs[q8], sem_i[q])

    def wait_inputs(bh, q8):
        q = q8 % D
        pltpu.make_async_copy(s_hbm.at[bh, :, pl.ds(i0, rows), :], s_vs[q],
                              sem_s[q]).wait()
        pltpu.make_async_copy(ipv_hbm.at[bh, :, pl.ds(ir0, irows), :],
                              ipv_vs[q8], sem_i[q]).wait()

    for bh0 in range(D):
        fire_inputs(bh0, bh0)

    c127 = jnp.full((16,), 127, jnp.int32)

    def bh_iter(bh, q8):
        q = q8 % D
        h = bh % H

        @pl.when(bh >= D)
        def _drain_and_rezero():
            pltpu.make_async_copy(w_vs[q],
                                  w_hbm.at[bh - D, :, pl.ds(i0, rows), :],
                                  sem_o[q]).wait()
            oidx = ipv_vs[(q8 + D) % 8]
            for r in range(rows):
                r16 = jnp.full((16,), r, jnp.int32)
                for g in range(ngroups):
                    og = slc(oidx, 0, r, g)
                    plsc.store_scatter(
                        w_vs[q],
                        [lax.shift_right_logical(og, 7), r16,
                         lax.bitwise_and(og, c127)],
                        zeros16)

        wait_inputs(bh, q8)
        hbase = jnp.full((16,), h * 2 * rows, jnp.int32)
        for r in range(rows):
            r16 = jnp.full((16,), r, jnp.int32)
            rbase = hbase + (2 * r)
            for g in range(ngroups):
                idx_g = slc(ipv_vs[q8], 0, r, g)
                jhi = lax.shift_right_logical(idx_g, 7)
                jlo = lax.bitwise_and(idx_g, c127)
                s_g = plsc.load_gather(s_vs[q], [jhi, r16, jlo])
                rb_g = plsc.load_gather(r_v, [rbase + jhi, jlo])
                pv_g = plsc.bitcast(slc(ipv_vs[q8], 1, r, g), jnp.float32)
                e_g = jnp.exp(s_g + rb_g) * pv_g
                plsc.addupdate_scatter(w_vs[q], [jhi, r16, jlo], e_g)
        pltpu.async_copy(w_vs[q], w_hbm.at[bh, :, pl.ds(i0, rows), :],
                         sem_o[q])

        @pl.when(bh + D < nbh)
        def _prefetch():
            fire_inputs(bh + D, (q8 + D) % 8)

    def outer(t, carry):
        for qq in range(8):
            bh_iter(t * 8 + qq, qq)
        return carry

    lax.fori_loop(0, nbh // 8, outer, 0)

    for dd in range(D):
        bh = nbh - D + dd
        pltpu.make_async_copy(w_vs[bh % D],
                              w_hbm.at[bh, :, pl.ds(i0, rows), :],
                              sem_o[bh % D]).wait()


def _make_sc_attn(b_, n, H, topk, nrpbf):
    rows = n // 32
    irows = (rows * topk) // 128
    mesh = plsc.VectorSubcoreMesh(core_axis_name="c", subcore_axis_name="s",
                                  num_cores=2, num_subcores=16)
    ipv_buf = pltpu.VMEM((2, irows, 128), jnp.int32)
    return pl.kernel(
        functools.partial(_sc_attn_body, b_=b_, n=n, H=H, topk=topk,
                          rows=rows),
        out_type=jax.ShapeDtypeStruct((b_ * H, 2, n, 128), jnp.float32),
        mesh=mesh,
        compiler_params=pltpu.CompilerParams(use_tc_tiling_on_sc=False,
                                             needs_layout_passes=False),
        scratch_types=(
            [pltpu.VMEM((2, rows, 128), jnp.float32)] * 4 +
            [ipv_buf] * 8 +
            [pltpu.VMEM((2, rows, 128), jnp.float32)] * 4 +
            [
                pltpu.VMEM((2 * rows, 128), jnp.int32),
                pltpu.VMEM((nrpbf, 128), jnp.float32),
                pltpu.VMEM((H * 2 * rows, 128), jnp.float32),
            ] +
            [pltpu.SemaphoreType.DMA] * 12
        ),
    )


def _out_kernel(w_ref, qkvp_ref, pw_ref, pb_ref, o_ref, *, c, H, dh):
    blk = qkvp_ref[0]
    parts = []
    for h in range(H):
        v = blk[:, h * dh:(h + 1) * dh]
        w0 = w_ref[h, 0]
        w1 = w_ref[h, 1]
        den = (jnp.sum(w0, axis=1, keepdims=True) +
               jnp.sum(w1, axis=1, keepdims=True) + 1e-20)
        xh = lax.dot_general(w0, v[0:128, :], (((1,), (0,)), ((), ())),
                             preferred_element_type=jnp.float32)
        xh = xh + lax.dot_general(w1, v[128:256, :],
                                  (((1,), (0,)), ((), ())),
                                  preferred_element_type=jnp.float32)
        parts.append(xh / den)
    x = jnp.concatenate(parts, axis=1) + blk[:, c:2 * c]
    o = lax.dot_general(x, pw_ref[...], (((1,), (1,)), ((), ())),
                        preferred_element_type=jnp.float32)
    o_ref[0] = o + pb_ref[...]


def _make_out(bc, boff, n, c, H, dh):
    return pl.pallas_call(
        functools.partial(_out_kernel, c=c, H=H, dh=dh),
        grid=(bc,),
        in_specs=[
            pl.BlockSpec((H, 2, n, 128), lambda b: (b, 0, 0, 0)),
            pl.BlockSpec((1, n, 2 * c), lambda b: (b + boff, 0, 1)),
            pl.BlockSpec((c, c), lambda b: (0, 0)),
            pl.BlockSpec((1, c), lambda b: (0, 0)),
        ],
        out_specs=pl.BlockSpec((1, n, c), lambda b: (b, 0, 0)),
        out_shape=jax.ShapeDtypeStruct((bc, n, c), jnp.float32),
    )


def kernel(qkvp, pfa_values, pfa_indices, rpi, rpb_table, proj_w, proj_b,
           shift):
    b_, n, c4 = qkvp.shape
    c = c4 // 4
    H = rpb_table.shape[1]
    dh = c // H
    topk = pfa_indices.shape[-1]
    nrpb = rpb_table.shape[0]
    nrpbf = -(-(nrpb * H) // 128)
    scale = dh ** (-0.5)

    shift_arr = jnp.asarray(shift, jnp.int32).reshape(1)
    rpir = rpi.astype(jnp.int32).reshape((n * n) // 128, 128)
    rpbf = jnp.pad(rpb_table.reshape(-1),
                   (0, nrpbf * 128 - nrpb * H)).reshape(nrpbf, 128)
    idx32 = pfa_indices.astype(jnp.int32)
    pb2 = proj_b.reshape(1, c)

    nchunks = 2
    bc = b_ // nchunks
    outs = []
    for ci in range(nchunks):
        boff = ci * bc
        s, ipv = _make_scores(bc, boff, n, c, H, dh, topk, scale)(
            shift_arr, qkvp, idx32, pfa_values)
        w = _make_sc_attn(bc, n, H, topk, nrpbf)(s, ipv, rpir, rpbf)
        outs.append(_make_out(bc, boff, n, c, H, dh)(w, qkvp, proj_w, pb2))
    return jnp.concatenate(outs, axis=0)


# speedup vs baseline: 1.1603x; 1.0819x over previous
import functools

import jax
import jax.numpy as jnp
from jax import lax
from jax.experimental import pallas as pl
from jax.experimental.pallas import tpu as pltpu
from jax.experimental.pallas import tpu_sc as plsc


def _scores_kernel(shift_ref, qkvp_ref, idx_ref, pv_ref,
                   s_ref, io_ref, *, c, H, dh, scale, n, topk):
    blk = qkvp_ref[0]
    for h in range(H):
        q = blk[:, h * dh:(h + 1) * dh]
        k = blk[:, c + h * dh:c + (h + 1) * dh]
        for par in range(2):
            kp = k[par * 128:(par + 1) * 128, :]
            s = lax.dot_general(q, kp, (((1,), (1,)), ((), ())),
                                preferred_element_type=jnp.float32)
            s_ref[h, par] = s * scale
    for h in range(H):
        for m in range(128 // topk):
            io_ref[h, 0, :, m * topk:(m + 1) * topk] = (
                idx_ref[0, 0, h, m::(128 // topk), :])
            io_ref[h, 1, :, m * topk:(m + 1) * topk] = (
                lax.bitcast_convert_type(
                    pv_ref[0, 0, h, m::(128 // topk), :], jnp.int32))


def _make_scores(bc, boff, n, c, H, dh, topk, scale):
    nr = (n * topk) // 128
    return pl.pallas_call(
        functools.partial(_scores_kernel, c=c, H=H, dh=dh, scale=scale,
                          n=n, topk=topk),
        grid_spec=pltpu.PrefetchScalarGridSpec(
            num_scalar_prefetch=1,
            grid=(bc,),
            in_specs=[
                pl.BlockSpec((1, n, 2 * c), lambda b, sref: (b + boff, 0, 0)),
                pl.BlockSpec((1, 1, H, n, topk),
                             lambda b, sref: (sref[0], b + boff, 0, 0, 0)),
                pl.BlockSpec((1, 1, H, n, topk),
                             lambda b, sref: (sref[0], b + boff, 0, 0, 0)),
            ],
            out_specs=[
                pl.BlockSpec((H, 2, n, 128), lambda b, sref: (b, 0, 0, 0)),
                pl.BlockSpec((H, 2, nr, 128), lambda b, sref: (b, 0, 0, 0)),
            ],
        ),
        out_shape=[
            jax.ShapeDtypeStruct((bc * H, 2, n, 128), jnp.float32),
            jax.ShapeDtypeStruct((bc * H, 2, nr, 128), jnp.int32),
        ],
    )


def _sc_attn_body(s_hbm, ipv_hbm, rpi_hbm, rpb_hbm, w_hbm,
                  s_v0, s_v1, s_v2, s_v3,
                  ix0, ix1, ix2, ix3, ix4, ix5, ix6, ix7,
                  w_v0, w_v1, w_v2, w_v3, rpi_v, rpb_v, r_v,
                  sem_s0, sem_s1, sem_s2, sem_s3,
                  sem_i0, sem_i1, sem_i2, sem_i3,
                  sem_o0, sem_o1, sem_o2, sem_o3,
                  *, b_, n, H, topk, rows):
    cid = lax.axis_index("c")
    sid = lax.axis_index("s")
    wid = sid * 2 + cid
    i0 = wid * rows

    s_vs = [s_v0, s_v1, s_v2, s_v3]
    ipv_vs = [ix0, ix1, ix2, ix3, ix4, ix5, ix6, ix7]
    w_vs = [w_v0, w_v1, w_v2, w_v3]
    sem_s = [sem_s0, sem_s1, sem_s2, sem_s3]
    sem_i = [sem_i0, sem_i1, sem_i2, sem_i3]
    sem_o = [sem_o0, sem_o1, sem_o2, sem_o3]

    pltpu.sync_copy(rpi_hbm.at[pl.ds(i0 * 2, rows * 2), :], rpi_v)
    pltpu.sync_copy(rpb_hbm, rpb_v)

    for h in range(H):
        h16 = jnp.full((16,), h, jnp.int32)
        for rr in range(2 * rows):
            for g in range(128 // 16):
                ri = rpi_v[rr, g * 16:(g + 1) * 16]
                f = ri * H + h16
                rb = plsc.load_gather(
                    rpb_v, [lax.shift_right_logical(f, 7),
                            lax.bitwise_and(f, jnp.full((16,), 127,
                                                        jnp.int32))])
                r_v[h * 2 * rows + rr, g * 16:(g + 1) * 16] = rb

    zeros16 = jnp.zeros((16,), jnp.float32)
    for p in range(4):
        for par in range(2):
            for r in range(rows):
                for j0 in range(0, 128, 16):
                    w_vs[p][par, r, j0:j0 + 16] = zeros16

    ngroups = topk // 16
    nbh = b_ * H
    ir0 = (i0 * topk) // 128
    irows = (rows * topk) // 128

    def slc(buf, pl_, r, g):
        flat = r * topk + g * 16
        return buf[pl_, flat // 128, flat % 128:flat % 128 + 16]

    D = 2

    def fire_inputs(bh, q8):
        q = q8 % D
        pltpu.async_copy(s_hbm.at[bh, :, pl.ds(i0, rows), :], s_vs[q],
                         sem_s[q])
        pltpu.async_copy(ipv_hbm.at[bh, :, pl.ds(ir0, irows), :],
                         ipv_vs[q8], sem_i[q])

    def wait_inputs(bh, q8):
        q = q8 % D
        pltpu.make_async_copy(s_hbm.at[bh, :, pl.ds(i0, rows), :], s_vs[q],
                              sem_s[q]).wait()
        pltpu.make_async_copy(ipv_hbm.at[bh, :, pl.ds(ir0, irows), :],
                              ipv_vs[q8], sem_i[q]).wait()

    for bh0 in range(D):
        fire_inputs(bh0, bh0)

    c127 = jnp.full((16,), 127, jnp.int32)

    def bh_iter(bh, q8):
        q = q8 % D
        h = bh % H

        @pl.when(bh >= D)
        def _drain_and_rezero():
            pltpu.make_async_copy(w_vs[q],
                                  w_hbm.at[bh - D, :, pl.ds(i0, rows), :],
                                  sem_o[q]).wait()
            oidx = ipv_vs[(q8 + D) % 4]
            for r in range(rows):
                r16 = jnp.full((16,), r, jnp.int32)
                for g in range(ngroups):
                    og = slc(oidx, 0, r, g)
                    plsc.store_scatter(
                        w_vs[q],
                        [lax.shift_right_logical(og, 7), r16,
                         lax.bitwise_and(og, c127)],
                        zeros16)

        wait_inputs(bh, q8)
        hbase = jnp.full((16,), h * 2 * rows, jnp.int32)
        for r in range(rows):
            r16 = jnp.full((16,), r, jnp.int32)
            rbase = hbase + (2 * r)
            for g in range(ngroups):
                idx_g = slc(ipv_vs[q8], 0, r, g)
                jhi = lax.shift_right_logical(idx_g, 7)
                jlo = lax.bitwise_and(idx_g, c127)
                s_g = plsc.load_gather(s_vs[q], [jhi, r16, jlo])
                rb_g = plsc.load_gather(r_v, [rbase + jhi, jlo])
                pv_g = plsc.bitcast(slc(ipv_vs[q8], 1, r, g), jnp.float32)
                e_g = jnp.exp(s_g + rb_g) * pv_g
                plsc.addupdate_scatter(w_vs[q], [jhi, r16, jlo], e_g)
        pltpu.async_copy(w_vs[q], w_hbm.at[bh, :, pl.ds(i0, rows), :],
                         sem_o[q])

        @pl.when(bh + D < nbh)
        def _prefetch():
            fire_inputs(bh + D, (q8 + D) % 4)

    def outer(t, carry):
        for qq in range(4):
            bh_iter(t * 4 + qq, qq)
        return carry

    lax.fori_loop(0, nbh // 4, outer, 0)

    for dd in range(D):
        bh = nbh - D + dd
        pltpu.make_async_copy(w_vs[bh % D],
                              w_hbm.at[bh, :, pl.ds(i0, rows), :],
                              sem_o[bh % D]).wait()


def _make_sc_attn(b_, n, H, topk, nrpbf):
    rows = n // 32
    irows = (rows * topk) // 128
    mesh = plsc.VectorSubcoreMesh(core_axis_name="c", subcore_axis_name="s",
                                  num_cores=2, num_subcores=16)
    ipv_buf = pltpu.VMEM((2, irows, 128), jnp.int32)
    return pl.kernel(
        functools.partial(_sc_attn_body, b_=b_, n=n, H=H, topk=topk,
                          rows=rows),
        out_type=jax.ShapeDtypeStruct((b_ * H, 2, n, 128), jnp.float32),
        mesh=mesh,
        compiler_params=pltpu.CompilerParams(use_tc_tiling_on_sc=False,
                                             needs_layout_passes=False),
        scratch_types=(
            [pltpu.VMEM((2, rows, 128), jnp.float32)] * 4 +
            [ipv_buf] * 8 +
            [pltpu.VMEM((2, rows, 128), jnp.float32)] * 4 +
            [
                pltpu.VMEM((2 * rows, 128), jnp.int32),
                pltpu.VMEM((nrpbf, 128), jnp.float32),
                pltpu.VMEM((H * 2 * rows, 128), jnp.float32),
            ] +
            [pltpu.SemaphoreType.DMA] * 12
        ),
    )


def _out_kernel(w_ref, qkvp_ref, pw_ref, pb_ref, o_ref, *, c, H, dh):
    blk = qkvp_ref[0]
    parts = []
    for h in range(H):
        v = blk[:, h * dh:(h + 1) * dh]
        w0 = w_ref[h, 0]
        w1 = w_ref[h, 1]
        den = (jnp.sum(w0, axis=1, keepdims=True) +
               jnp.sum(w1, axis=1, keepdims=True) + 1e-20)
        xh = lax.dot_general(w0, v[0:128, :], (((1,), (0,)), ((), ())),
                             preferred_element_type=jnp.float32)
        xh = xh + lax.dot_general(w1, v[128:256, :],
                                  (((1,), (0,)), ((), ())),
                                  preferred_element_type=jnp.float32)
        parts.append(xh / den)
    x = jnp.concatenate(parts, axis=1) + blk[:, c:2 * c]
    o = lax.dot_general(x, pw_ref[...], (((1,), (1,)), ((), ())),
                        preferred_element_type=jnp.float32)
    o_ref[0] = o + pb_ref[...]


def _make_out(bc, boff, n, c, H, dh):
    return pl.pallas_call(
        functools.partial(_out_kernel, c=c, H=H, dh=dh),
        grid=(bc,),
        in_specs=[
            pl.BlockSpec((H, 2, n, 128), lambda b: (b, 0, 0, 0)),
            pl.BlockSpec((1, n, 2 * c), lambda b: (b + boff, 0, 1)),
            pl.BlockSpec((c, c), lambda b: (0, 0)),
            pl.BlockSpec((1, c), lambda b: (0, 0)),
        ],
        out_specs=pl.BlockSpec((1, n, c), lambda b: (b, 0, 0)),
        out_shape=jax.ShapeDtypeStruct((bc, n, c), jnp.float32),
    )


def kernel(qkvp, pfa_values, pfa_indices, rpi, rpb_table, proj_w, proj_b,
           shift):
    b_, n, c4 = qkvp.shape
    c = c4 // 4
    H = rpb_table.shape[1]
    dh = c // H
    topk = pfa_indices.shape[-1]
    nrpb = rpb_table.shape[0]
    nrpbf = -(-(nrpb * H) // 128)
    scale = dh ** (-0.5)

    shift_arr = jnp.asarray(shift, jnp.int32).reshape(1)
    rpir = rpi.astype(jnp.int32).reshape((n * n) // 128, 128)
    rpbf = jnp.pad(rpb_table.reshape(-1),
                   (0, nrpbf * 128 - nrpb * H)).reshape(nrpbf, 128)
    idx32 = pfa_indices.astype(jnp.int32)
    pb2 = proj_b.reshape(1, c)

    nchunks = 2
    bc = b_ // nchunks
    outs = []
    for ci in range(nchunks):
        boff = ci * bc
        s, ipv = _make_scores(bc, boff, n, c, H, dh, topk, scale)(
            shift_arr, qkvp, idx32, pfa_values)
        w = _make_sc_attn(bc, n, H, topk, nrpbf)(s, ipv, rpir, rpbf)
        outs.append(_make_out(bc, boff, n, c, H, dh)(w, qkvp, proj_w, pb2))
    return jnp.concatenate(outs, axis=0)


# speedup vs baseline: 1.1773x; 1.0146x over previous
import functools

import jax
import jax.numpy as jnp
from jax import lax
from jax.experimental import pallas as pl
from jax.experimental.pallas import tpu as pltpu
from jax.experimental.pallas import tpu_sc as plsc


def _scores_kernel(shift_ref, qkvp_ref, idx_ref, pv_ref,
                   s_ref, io_ref, *, c, H, dh, scale, n, topk):
    blk = qkvp_ref[0]
    for h in range(H):
        q = blk[:, h * dh:(h + 1) * dh]
        k = blk[:, c + h * dh:c + (h + 1) * dh]
        for par in range(2):
            kp = k[par * 128:(par + 1) * 128, :]
            s = lax.dot_general(q, kp, (((1,), (1,)), ((), ())),
                                preferred_element_type=jnp.float32)
            s_ref[h, par] = s * scale
    for h in range(H):
        for m in range(128 // topk):
            io_ref[h, 0, :, m * topk:(m + 1) * topk] = (
                idx_ref[0, 0, h, m::(128 // topk), :])
            io_ref[h, 1, :, m * topk:(m + 1) * topk] = (
                lax.bitcast_convert_type(
                    pv_ref[0, 0, h, m::(128 // topk), :], jnp.int32))


def _make_scores(bc, boff, n, c, H, dh, topk, scale):
    nr = (n * topk) // 128
    return pl.pallas_call(
        functools.partial(_scores_kernel, c=c, H=H, dh=dh, scale=scale,
                          n=n, topk=topk),
        grid_spec=pltpu.PrefetchScalarGridSpec(
            num_scalar_prefetch=1,
            grid=(bc,),
            in_specs=[
                pl.BlockSpec((1, n, 2 * c), lambda b, sref: (b + boff, 0, 0)),
                pl.BlockSpec((1, 1, H, n, topk),
                             lambda b, sref: (sref[0], b + boff, 0, 0, 0)),
                pl.BlockSpec((1, 1, H, n, topk),
                             lambda b, sref: (sref[0], b + boff, 0, 0, 0)),
            ],
            out_specs=[
                pl.BlockSpec((H, 2, n, 128), lambda b, sref: (b, 0, 0, 0)),
                pl.BlockSpec((H, 2, nr, 128), lambda b, sref: (b, 0, 0, 0)),
            ],
        ),
        out_shape=[
            jax.ShapeDtypeStruct((bc * H, 2, n, 128), jnp.float32),
            jax.ShapeDtypeStruct((bc * H, 2, nr, 128), jnp.int32),
        ],
    )


def _sc_attn_body(s_hbm, ipv_hbm, rpi_hbm, rpb_hbm, w_hbm,
                  s_v0, s_v1, s_v2, s_v3,
                  ix0, ix1, ix2, ix3, ix4, ix5, ix6, ix7,
                  w_v0, w_v1, w_v2, w_v3, rpi_v, rpb_v, r_v,
                  sem_s0, sem_s1, sem_s2, sem_s3,
                  sem_i0, sem_i1, sem_i2, sem_i3,
                  sem_o0, sem_o1, sem_o2, sem_o3,
                  *, b_, n, H, topk, rows):
    cid = lax.axis_index("c")
    sid = lax.axis_index("s")
    wid = sid * 2 + cid
    i0 = wid * rows

    s_vs = [s_v0, s_v1, s_v2, s_v3]
    ipv_vs = [ix0, ix1, ix2, ix3, ix4, ix5, ix6, ix7]
    w_vs = [w_v0, w_v1, w_v2, w_v3]
    sem_s = [sem_s0, sem_s1, sem_s2, sem_s3]
    sem_i = [sem_i0, sem_i1, sem_i2, sem_i3]
    sem_o = [sem_o0, sem_o1, sem_o2, sem_o3]

    pltpu.sync_copy(rpi_hbm.at[pl.ds(i0 * 2, rows * 2), :], rpi_v)
    pltpu.sync_copy(rpb_hbm, rpb_v)

    for h in range(H):
        h16 = jnp.full((16,), h, jnp.int32)
        for rr in range(2 * rows):
            for g in range(128 // 16):
                ri = rpi_v[rr, g * 16:(g + 1) * 16]
                f = ri * H + h16
                rb = plsc.load_gather(
                    rpb_v, [lax.shift_right_logical(f, 7),
                            lax.bitwise_and(f, jnp.full((16,), 127,
                                                        jnp.int32))])
                r_v[h * 2 * rows + rr, g * 16:(g + 1) * 16] = rb

    zeros16 = jnp.zeros((16,), jnp.float32)
    for p in range(2):
        for par in range(2):
            for r in range(rows):
                for j0 in range(0, 128, 16):
                    w_vs[p][par, r, j0:j0 + 16] = zeros16

    ngroups = topk // 16
    nbh = b_ * H
    ir0 = (i0 * topk) // 128
    irows = (rows * topk) // 128

    def slc(buf, pl_, r, g):
        flat = r * topk + g * 16
        return buf[pl_, flat // 128, flat % 128:flat % 128 + 16]

    D = 2

    def fire_inputs(bh, q8):
        q = q8 % D
        pltpu.async_copy(s_hbm.at[bh, :, pl.ds(i0, rows), :], s_vs[q],
                         sem_s[q])
        pltpu.async_copy(ipv_hbm.at[bh, :, pl.ds(ir0, irows), :],
                         ipv_vs[q8], sem_i[q])

    def wait_inputs(bh, q8):
        q = q8 % D
        pltpu.make_async_copy(s_hbm.at[bh, :, pl.ds(i0, rows), :], s_vs[q],
                              sem_s[q]).wait()
        pltpu.make_async_copy(ipv_hbm.at[bh, :, pl.ds(ir0, irows), :],
                              ipv_vs[q8], sem_i[q]).wait()

    for bh0 in range(D):
        fire_inputs(bh0, bh0)

    c127 = jnp.full((16,), 127, jnp.int32)

    def bh_iter(bh, q8):
        q = q8 % D
        h = bh % H

        @pl.when(bh >= D)
        def _drain_and_rezero():
            pltpu.make_async_copy(w_vs[q],
                                  w_hbm.at[bh - D, :, pl.ds(i0, rows), :],
                                  sem_o[q]).wait()
            oidx = ipv_vs[(q8 + D) % 4]
            for r in range(rows):
                r16 = jnp.full((16,), r, jnp.int32)
                for g in range(ngroups):
                    og = slc(oidx, 0, r, g)
                    plsc.store_scatter(
                        w_vs[q],
                        [lax.shift_right_logical(og, 7), r16,
                         lax.bitwise_and(og, c127)],
                        zeros16)

        wait_inputs(bh, q8)
        hbase = jnp.full((16,), h * 2 * rows, jnp.int32)
        for r in range(rows):
            r16 = jnp.full((16,), r, jnp.int32)
            rbase = hbase + (2 * r)
            for g in range(ngroups):
                idx_g = slc(ipv_vs[q8], 0, r, g)
                jhi = lax.shift_right_logical(idx_g, 7)
                jlo = lax.bitwise_and(idx_g, c127)
                s_g = plsc.load_gather(s_vs[q], [jhi, r16, jlo])
                rb_g = plsc.load_gather(r_v, [rbase + jhi, jlo])
                pv_g = plsc.bitcast(slc(ipv_vs[q8], 1, r, g), jnp.float32)
                e_g = jnp.exp(s_g + rb_g) * pv_g
                plsc.addupdate_scatter(w_vs[q], [jhi, r16, jlo], e_g)
        pltpu.async_copy(w_vs[q], w_hbm.at[bh, :, pl.ds(i0, rows), :],
                         sem_o[q])

        @pl.when(bh + D < nbh)
        def _prefetch():
            fire_inputs(bh + D, (q8 + D) % 4)

    def outer(t, carry):
        for qq in range(4):
            bh_iter(t * 4 + qq, qq)
        return carry

    lax.fori_loop(0, nbh // 4, outer, 0)

    for dd in range(D):
        bh = nbh - D + dd
        pltpu.make_async_copy(w_vs[bh % D],
                              w_hbm.at[bh, :, pl.ds(i0, rows), :],
                              sem_o[bh % D]).wait()


def _make_sc_attn(b_, n, H, topk, nrpbf):
    rows = n // 32
    irows = (rows * topk) // 128
    mesh = plsc.VectorSubcoreMesh(core_axis_name="c", subcore_axis_name="s",
                                  num_cores=2, num_subcores=16)
    ipv_buf = pltpu.VMEM((2, irows, 128), jnp.int32)
    return pl.kernel(
        functools.partial(_sc_attn_body, b_=b_, n=n, H=H, topk=topk,
                          rows=rows),
        out_type=jax.ShapeDtypeStruct((b_ * H, 2, n, 128), jnp.float32),
        mesh=mesh,
        compiler_params=pltpu.CompilerParams(use_tc_tiling_on_sc=False,
                                             needs_layout_passes=False),
        scratch_types=(
            [pltpu.VMEM((2, rows, 128), jnp.float32)] * 4 +
            [ipv_buf] * 8 +
            [pltpu.VMEM((2, rows, 128), jnp.float32)] * 4 +
            [
                pltpu.VMEM((2 * rows, 128), jnp.int32),
                pltpu.VMEM((nrpbf, 128), jnp.float32),
                pltpu.VMEM((H * 2 * rows, 128), jnp.float32),
            ] +
            [pltpu.SemaphoreType.DMA] * 12
        ),
    )


def _out_kernel(w_ref, qkvp_ref, pw_ref, pb_ref, o_ref, *, c, H, dh):
    blk = qkvp_ref[0]
    parts = []
    for h in range(H):
        v = blk[:, h * dh:(h + 1) * dh]
        w0 = w_ref[h, 0]
        w1 = w_ref[h, 1]
        den = (jnp.sum(w0, axis=1, keepdims=True) +
               jnp.sum(w1, axis=1, keepdims=True) + 1e-20)
        xh = lax.dot_general(w0, v[0:128, :], (((1,), (0,)), ((), ())),
                             preferred_element_type=jnp.float32)
        xh = xh + lax.dot_general(w1, v[128:256, :],
                                  (((1,), (0,)), ((), ())),
                                  preferred_element_type=jnp.float32)
        parts.append(xh / den)
    x = jnp.concatenate(parts, axis=1) + blk[:, c:2 * c]
    o = lax.dot_general(x, pw_ref[...], (((1,), (1,)), ((), ())),
                        preferred_element_type=jnp.float32)
    o_ref[0] = o + pb_ref[...]


def _out_kernel_chained(w_ref, qkvp_ref, pw_ref, pb_ref, prev_ref, o_ref,
                        *, c, H, dh):
    del prev_ref
    _out_kernel(w_ref, qkvp_ref, pw_ref, pb_ref, o_ref, c=c, H=H, dh=dh)


def _make_out(bc, boff, b_, n, c, H, dh, chained):
    in_specs = [
        pl.BlockSpec((H, 2, n, 128), lambda b: (b, 0, 0, 0)),
        pl.BlockSpec((1, n, 2 * c), lambda b: (b + boff, 0, 1)),
        pl.BlockSpec((c, c), lambda b: (0, 0)),
        pl.BlockSpec((1, c), lambda b: (0, 0)),
    ]
    kwargs = {}
    body = _out_kernel
    if chained:
        in_specs.append(pl.BlockSpec(memory_space=pl.ANY))
        kwargs["input_output_aliases"] = {4: 0}
        body = _out_kernel_chained
    return pl.pallas_call(
        functools.partial(body, c=c, H=H, dh=dh),
        grid=(bc,),
        in_specs=in_specs,
        out_specs=pl.BlockSpec((1, n, c), lambda b: (b + boff, 0, 0)),
        out_shape=jax.ShapeDtypeStruct((b_, n, c), jnp.float32),
        **kwargs,
    )


def kernel(qkvp, pfa_values, pfa_indices, rpi, rpb_table, proj_w, proj_b,
           shift):
    b_, n, c4 = qkvp.shape
    c = c4 // 4
    H = rpb_table.shape[1]
    dh = c // H
    topk = pfa_indices.shape[-1]
    nrpb = rpb_table.shape[0]
    nrpbf = -(-(nrpb * H) // 128)
    scale = dh ** (-0.5)

    shift_arr = jnp.asarray(shift, jnp.int32).reshape(1)
    rpir = rpi.astype(jnp.int32).reshape((n * n) // 128, 128)
    rpbf = jnp.pad(rpb_table.reshape(-1),
                   (0, nrpbf * 128 - nrpb * H)).reshape(nrpbf, 128)
    idx32 = pfa_indices.astype(jnp.int32)
    pb2 = proj_b.reshape(1, c)

    nchunks = 2
    bc = b_ // nchunks
    out = None
    for ci in range(nchunks):
        boff = ci * bc
        s, ipv = _make_scores(bc, boff, n, c, H, dh, topk, scale)(
            shift_arr, qkvp, idx32, pfa_values)
        w = _make_sc_attn(bc, n, H, topk, nrpbf)(s, ipv, rpir, rpbf)
        mk = _make_out(bc, boff, b_, n, c, H, dh, chained=ci > 0)
        if ci == 0:
            out = mk(w, qkvp, proj_w, pb2)
        else:
            out = mk(w, qkvp, proj_w, pb2, out)
    return out
